# baseline (speedup 1.0000x reference)
import jax
import jax.numpy as jnp
from jax import lax
from jax.experimental import pallas as pl
from jax.experimental.pallas import tpu as pltpu
from jax.experimental.pallas import tpu_sc as plsc

BATCH = 4096
HIST = 200
D_MODEL = 64
D_PAD = 128
B_TOTAL = BATCH * HIST

NUM_CORES = 2
NUM_SUBCORES = 16
NW = NUM_CORES * NUM_SUBCORES
B_PER_W = B_TOTAL // NW

CH = 128
NB = 2
NCH = B_PER_W // CH


def _gather_body(idx_hbm, table_hbm, out_hbm, *scratch):
    idx_v = scratch[:NB]
    rows_v = scratch[NB : 2 * NB]
    pack_v = scratch[2 * NB : 3 * NB]
    shared_v = scratch[3 * NB]
    gsem = scratch[3 * NB + 1 : 4 * NB + 1]
    osem = scratch[4 * NB + 1 :]
    sid = lax.axis_index("s")
    wid = sid * NUM_CORES + lax.axis_index("c")
    base = wid * B_PER_W

    def fire_gather(g, s):
        off = base + g * CH
        pltpu.sync_copy(idx_hbm.at[pl.ds(off, CH)], idx_v[s])
        pltpu.async_copy(table_hbm.at[idx_v[s]], rows_v[s], gsem[s])

    def wait_gather(s):
        pltpu.make_async_copy(
            table_hbm.at[idx_v[s]], rows_v[s], gsem[s]
        ).wait()

    def fire_out(g, s):
        off = base + g * CH
        pltpu.sync_copy(pack_v[s], shared_v.at[sid, s])
        pltpu.async_copy(
            shared_v.at[sid, s], out_hbm.at[pl.ds(off, CH)], osem[s]
        )

    def wait_out(g, s):
        off = base + g * CH
        pltpu.make_async_copy(
            shared_v.at[sid, s], out_hbm.at[pl.ds(off, CH)], osem[s]
        ).wait()

    def compact(s):
        @pl.loop(0, CH, step=4)
        def _row(j):
            for jj in range(4):
                for k in range(D_MODEL // 16):
                    pack_v[s][j + jj, pl.ds(16 * k, 16)] = rows_v[s][
                        j + jj, pl.ds(16 * k, 16)
                    ]

    for s in range(NB):
        fire_gather(s, s)
    for s in range(NB):
        wait_gather(s)
        compact(s)
        fire_out(s, s)
        fire_gather(s + NB, s)

    @pl.loop(NB, NCH - NB, step=NB)
    def _ring(p):
        for s in range(NB):
            g = p + s
            wait_gather(s)
            wait_out(g - NB, s)
            compact(s)
            fire_gather(g + NB, s)
            fire_out(g, s)

    for s in range(NB):
        g = NCH - NB + s
        wait_gather(s)
        wait_out(g - NB, s)
        compact(s)
        fire_out(g, s)
    for s in range(NB):
        wait_out(NCH - NB + s, s)


@jax.jit
def _embed_lookup(idx_flat, table128):
    mesh = plsc.VectorSubcoreMesh(core_axis_name="c", subcore_axis_name="s")
    grid_kernel = pl.kernel(
        _gather_body,
        out_type=jax.ShapeDtypeStruct((B_TOTAL, D_MODEL), jnp.float32),
        mesh=mesh,
        scratch_types=[pltpu.VMEM((CH,), jnp.int32)] * NB
        + [pltpu.VMEM((CH, D_PAD), jnp.float32)] * NB
        + [pltpu.VMEM((CH, D_MODEL), jnp.float32)] * NB
        + [
            pltpu.VMEM_SHARED(
                (NUM_SUBCORES, NB, CH, D_MODEL), jnp.float32
            )
        ]
        + [pltpu.SemaphoreType.DMA] * (2 * NB),
    )
    return grid_kernel(idx_flat, table128)


def kernel(x, order, embed_weight):
    idx_flat = x.reshape(B_TOTAL).astype(jnp.int32)
    table128 = jnp.pad(embed_weight, ((0, 0), (0, D_PAD - D_MODEL)))
    out = _embed_lookup(idx_flat, table128)
    return out.reshape(BATCH, HIST, D_MODEL)


# speedup vs baseline: 1.0138x; 1.0138x over previous
import jax
import jax.numpy as jnp
from jax import lax
from jax.experimental import pallas as pl
from jax.experimental.pallas import tpu as pltpu
from jax.experimental.pallas import tpu_sc as plsc

BATCH = 4096
HIST = 200
D_MODEL = 64
D_PAD = 128
B_TOTAL = BATCH * HIST

NUM_CORES = 2
NUM_SUBCORES = 16
NW = NUM_CORES * NUM_SUBCORES
B_PER_W = B_TOTAL // NW

CH = 128
NB = 2
NCH = B_PER_W // CH


def _gather_body(idx_hbm, table_hbm, out_hbm, *scratch):
    idx_v = scratch[:NB]
    rows_v = scratch[NB : 2 * NB]
    pack_v = scratch[2 * NB : 3 * NB]
    gsem = scratch[3 * NB : 4 * NB]
    osem = scratch[4 * NB :]
    wid = lax.axis_index("s") * NUM_CORES + lax.axis_index("c")
    base = wid * B_PER_W

    def fire_gather(g, s):
        off = base + g * CH
        pltpu.sync_copy(idx_hbm.at[pl.ds(off, CH)], idx_v[s])
        pltpu.async_copy(table_hbm.at[idx_v[s]], rows_v[s], gsem[s])

    def wait_gather(s):
        pltpu.make_async_copy(
            table_hbm.at[idx_v[s]], rows_v[s], gsem[s]
        ).wait()

    def fire_out(g, s):
        off = base + g * CH
        pltpu.async_copy(pack_v[s], out_hbm.at[pl.ds(off, CH)], osem[s])

    def wait_out(g, s):
        off = base + g * CH
        pltpu.make_async_copy(
            pack_v[s], out_hbm.at[pl.ds(off, CH)], osem[s]
        ).wait()

    def compact(s):
        @pl.loop(0, CH, step=4)
        def _row(j):
            for jj in range(4):
                for k in range(D_MODEL // 16):
                    pack_v[s][j + jj, pl.ds(16 * k, 16)] = rows_v[s][
                        j + jj, pl.ds(16 * k, 16)
                    ]

    for s in range(NB):
        fire_gather(s, s)
    for s in range(NB):
        wait_gather(s)
        compact(s)
        fire_out(s, s)
        fire_gather(s + NB, s)

    @pl.loop(NB, NCH - NB, step=NB)
    def _ring(p):
        for s in range(NB):
            g = p + s
            wait_gather(s)
            wait_out(g - NB, s)
            compact(s)
            fire_gather(g + NB, s)
            fire_out(g, s)

    for s in range(NB):
        g = NCH - NB + s
        wait_gather(s)
        wait_out(g - NB, s)
        compact(s)
        fire_out(g, s)
    for s in range(NB):
        wait_out(NCH - NB + s, s)


@jax.jit
def _embed_lookup(idx_flat, table128):
    mesh = plsc.VectorSubcoreMesh(core_axis_name="c", subcore_axis_name="s")
    grid_kernel = pl.kernel(
        _gather_body,
        out_type=jax.ShapeDtypeStruct((B_TOTAL, D_MODEL), jnp.float32),
        mesh=mesh,
        scratch_types=[pltpu.VMEM((CH,), jnp.int32)] * NB
        + [pltpu.VMEM((CH, D_PAD), jnp.float32)] * NB
        + [pltpu.VMEM((CH, D_MODEL), jnp.float32)] * NB
        + [pltpu.SemaphoreType.DMA] * (2 * NB),
    )
    return grid_kernel(idx_flat, table128)


def kernel(x, order, embed_weight):
    idx_flat = x.reshape(B_TOTAL).astype(jnp.int32)
    table128 = jnp.pad(embed_weight, ((0, 0), (0, D_PAD - D_MODEL)))
    out = _embed_lookup(idx_flat, table128)
    return out.reshape(BATCH, HIST, D_MODEL)


# speedup vs baseline: 1.0480x; 1.0338x over previous
import jax
import jax.numpy as jnp
from jax import lax
from jax.experimental import pallas as pl
from jax.experimental.pallas import tpu as pltpu
from jax.experimental.pallas import tpu_sc as plsc

BATCH = 4096
HIST = 200
D_MODEL = 64
D_PAD = 128
B_TOTAL = BATCH * HIST

NUM_CORES = 2
NUM_SUBCORES = 16
NW = NUM_CORES * NUM_SUBCORES
B_PER_W = B_TOTAL // NW

CH = 128
NB = 2
NCH = B_PER_W // CH


def _gather_body(idx_hbm, table_hbm, out_hbm, *scratch):
    idx_all = scratch[0]
    rows_v = scratch[1 : NB + 1]
    pack_v = scratch[NB + 1 : 2 * NB + 1]
    gsem = scratch[2 * NB + 1 : 3 * NB + 1]
    osem = scratch[3 * NB + 1 :]
    wid = lax.axis_index("s") * NUM_CORES + lax.axis_index("c")
    base = wid * B_PER_W

    pltpu.sync_copy(idx_hbm.at[pl.ds(base, B_PER_W)], idx_all)

    def fire_gather(g, s):
        pltpu.async_copy(
            table_hbm.at[idx_all.at[pl.ds(g * CH, CH)]], rows_v[s], gsem[s]
        )

    def wait_gather(s):
        pltpu.make_async_copy(
            table_hbm.at[idx_all.at[pl.ds(0, CH)]], rows_v[s], gsem[s]
        ).wait()

    def fire_out(g, s):
        off = base + g * CH
        pltpu.async_copy(pack_v[s], out_hbm.at[pl.ds(off, CH)], osem[s])

    def wait_out(g, s):
        off = base + g * CH
        pltpu.make_async_copy(
            pack_v[s], out_hbm.at[pl.ds(off, CH)], osem[s]
        ).wait()

    def compact(s):
        @pl.loop(0, CH, step=4)
        def _row(j):
            for jj in range(4):
                for k in range(D_MODEL // 16):
                    pack_v[s][j + jj, pl.ds(16 * k, 16)] = rows_v[s][
                        j + jj, pl.ds(16 * k, 16)
                    ]

    for s in range(NB):
        fire_gather(s, s)
    for s in range(NB):
        wait_gather(s)
        compact(s)
        fire_out(s, s)
        fire_gather(s + NB, s)

    @pl.loop(NB, NCH - NB, step=NB)
    def _ring(p):
        for s in range(NB):
            g = p + s
            wait_gather(s)
            wait_out(g - NB, s)
            compact(s)
            fire_gather(g + NB, s)
            fire_out(g, s)

    for s in range(NB):
        g = NCH - NB + s
        wait_gather(s)
        wait_out(g - NB, s)
        compact(s)
        fire_out(g, s)
    for s in range(NB):
        wait_out(NCH - NB + s, s)


@jax.jit
def _embed_lookup(idx_flat, table128):
    mesh = plsc.VectorSubcoreMesh(core_axis_name="c", subcore_axis_name="s")
    grid_kernel = pl.kernel(
        _gather_body,
        out_type=jax.ShapeDtypeStruct((B_TOTAL, D_MODEL), jnp.float32),
        mesh=mesh,
        scratch_types=[pltpu.VMEM((B_PER_W,), jnp.int32)]
        + [pltpu.VMEM((CH, D_PAD), jnp.float32)] * NB
        + [pltpu.VMEM((CH, D_MODEL), jnp.float32)] * NB
        + [pltpu.SemaphoreType.DMA] * (2 * NB),
    )
    return grid_kernel(idx_flat, table128)


def kernel(x, order, embed_weight):
    idx_flat = x.reshape(B_TOTAL).astype(jnp.int32)
    table128 = jnp.pad(embed_weight, ((0, 0), (0, D_PAD - D_MODEL)))
    out = _embed_lookup(idx_flat, table128)
    return out.reshape(BATCH, HIST, D_MODEL)
